# baseline (speedup 1.0000x reference)
import jax
import jax.numpy as jnp
from jax.experimental import pallas as pl
from jax.experimental.pallas import tpu as pltpu

B, C, L, D = 16384, 26, 128, 64
TAU = 0.07
MARGIN = 0.3

BB = 512
NB = B // BB


def _body(logc_ref, labc_ref, tlt_ref, tft_ref, pro_ref, out_ref):
    c = pl.program_id(1)
    x = logc_ref[0, 0]
    y = labc_ref[0, 0]
    t8 = tlt_ref[0, :, :8]
    f8 = tft_ref[0, :, :8]
    contrib = (x + y + jnp.max(t8, axis=1)[None, :]
               + jnp.max(f8, axis=1)[None, :] + jnp.sum(pro_ref[:, :8]))[0]

    @pl.when(c == 0)
    def _init():
        out_ref[...] = contrib

    @pl.when(c != 0)
    def _acc():
        out_ref[...] = out_ref[...] + contrib
    return


def _body_unused(logc_ref, labc_ref, tlt_ref, tft_ref, pro_ref, out_ref):
    c = pl.program_id(1)

    x = logc_ref[0, 0]
    y = labc_ref[0, 0]

    bce = jnp.maximum(x, 0.0) - x * y + jnp.log1p(jnp.exp(-jnp.abs(x)))

    t = tlt_ref[0]
    mxr = jnp.max(t, axis=1)
    rej = jnp.maximum(jax.nn.sigmoid(mxr) - MARGIN, 0.0)[None, :]

    p = pro_ref[...]
    pinv = 1.0 / jnp.maximum(
        jnp.sqrt(jnp.sum(p * p, axis=1, keepdims=True)), 1e-12)
    pn = p * pinv
    f = tft_ref[0]
    sqv = jax.lax.dot_general(jnp.ones((1, D), jnp.float32), f * f,
                              (((1,), (1,)), ((), ())),
                              preferred_element_type=jnp.float32)
    finv = 1.0 / jnp.maximum(jnp.sqrt(sqv), 1e-12)
    St = jax.lax.dot_general(pn, f, (((1,), (1,)), ((), ())),
                             preferred_element_type=jnp.float32)
    St = St * (finv * (1.0 / TAU))
    m = jnp.max(St, axis=0, keepdims=True)
    lse = m + jnp.log(jnp.sum(jnp.exp(St - m), axis=0, keepdims=True))
    row = jax.lax.broadcasted_iota(jnp.int32, (C, BB), 0)
    diag = jnp.sum(jnp.where(row == c, St, 0.0), axis=0, keepdims=True)
    psc = lse - diag

    contrib = (bce + jnp.where(y > 0.0, psc, rej))[0]

    @pl.when(c == 0)
    def _init():
        out_ref[...] = contrib

    @pl.when(c != 0)
    def _acc():
        out_ref[...] = out_ref[...] + contrib


def kernel(logits, total_cls_logits, total_cls_feature, labels, prototypes):
    logc = logits.T.reshape(C, NB, 1, BB)
    labc = labels.T.reshape(C, NB, 1, BB)
    grid = (NB, C)
    out = pl.pallas_call(
        _body,
        grid=grid,
        in_specs=[
            pl.BlockSpec((1, 1, 1, BB), lambda i, c: (c, i, 0, 0)),
            pl.BlockSpec((1, 1, 1, BB), lambda i, c: (c, i, 0, 0)),
            pl.BlockSpec((1, BB, L), lambda i, c: (c, i, 0)),
            pl.BlockSpec((1, BB, D), lambda i, c: (c, i, 0)),
            pl.BlockSpec((C, D), lambda i, c: (0, 0)),
        ],
        out_specs=pl.BlockSpec((BB,), lambda i, c: (i,)),
        out_shape=jax.ShapeDtypeStruct((B,), jnp.float32),
    )(logc, labc, total_cls_logits, total_cls_feature, prototypes)
    return out


# speedup vs baseline: 1.7217x; 1.7217x over previous
import functools

import jax
import jax.numpy as jnp
import numpy as np
from jax import lax
from jax.experimental import pallas as pl
from jax.experimental.pallas import tpu as pltpu
from jax.experimental.pallas import tpu_sc as plsc

B, C, L, D = 16384, 26, 128, 64
TAU = 0.07
MARGIN = 0.3


BB = 512
NB = B // BB

_OH = np.kron(np.eye(C, dtype=np.float32), np.ones((1, BB), np.float32))


def _tc_body(logT_ref, labT_ref, tft_ref, pro_ref, oh_ref, out_ref):
    x = logT_ref[...]
    y = labT_ref[...]

    bce = jnp.maximum(x, 0.0) - x * y + jnp.log1p(jnp.exp(-jnp.abs(x)))
    acc = jnp.sum(bce, axis=0)

    p = pro_ref[...]
    pinv = 1.0 / jnp.maximum(
        jnp.sqrt(jnp.sum(p * p, axis=1, keepdims=True)), 1e-12)
    pn = p * pinv
    f = tft_ref[...]
    F = f.reshape(C * BB, D)
    ones_row = jnp.ones((1, D), jnp.float32)
    sqv = jax.lax.dot_general(ones_row, F * F, (((1,), (1,)), ((), ())),
                              preferred_element_type=jnp.float32)
    finv = 1.0 / jnp.maximum(jnp.sqrt(sqv), 1e-12)
    St = jax.lax.dot_general(pn, F, (((1,), (1,)), ((), ())),
                             preferred_element_type=jnp.float32)
    St = St * (finv * (1.0 / TAU))
    m = jnp.max(St, axis=0, keepdims=True)
    lse = m + jnp.log(jnp.sum(jnp.exp(St - m), axis=0, keepdims=True))
    diag = jnp.sum(St * oh_ref[...], axis=0, keepdims=True)
    psc = (lse - diag).reshape(C, BB)
    acc = acc + jnp.sum(jnp.where(y > 0.0, psc, 0.0), axis=0)

    out_ref[...] = acc


def _tc_call(logT, labT, tft, pro, oh):
    return pl.pallas_call(
        _tc_body,
        grid=(NB,),
        in_specs=[
            pl.BlockSpec((C, BB), lambda i: (0, i)),
            pl.BlockSpec((C, BB), lambda i: (0, i)),
            pl.BlockSpec((C, BB, D), lambda i: (0, i, 0)),
            pl.BlockSpec((C, D), lambda i: (0, 0)),
            pl.BlockSpec((C, C * BB), lambda i: (0, 0)),
        ],
        out_specs=pl.BlockSpec((BB,), lambda i: (i,)),
        out_shape=jax.ShapeDtypeStruct((B,), jnp.float32),
    )(logT, labT, tft, pro, oh)


_NC = 2
_NS = 16
_NW = _NC * _NS
_SPW = B // _NW
_RPW = C * _SPW
_PAD = 64
_NEG = -1e30


def _sc_body(lab_hbm, tlt_hbm, out_hbm,
             lab_v, rid_v, bloc_v, bufa, bufb, acc_v, sema, semb):
    wid = lax.axis_index("s") * _NC + lax.axis_index("c")
    wbase = wid * _SPW

    iota16 = lax.iota(jnp.int32, 16)
    zero16 = jnp.zeros((16,), jnp.int32)

    def _init(i, _):
        rid_v[pl.ds(i * 16, 16)] = zero16
        bloc_v[pl.ds(i * 16, 16)] = zero16 + _SPW
        return 0
    lax.fori_loop(0, (_RPW + _PAD) // 16, _init, 0)

    def _initacc(i, _):
        acc_v[pl.ds(i * 16, 16)] = jnp.zeros((16,), jnp.float32)
        return 0
    lax.fori_loop(0, (_SPW + 16) // 16, _initacc, 0)

    pltpu.sync_copy(lab_hbm.at[pl.ds(wbase * C, _SPW * C)], lab_v)

    def _build(j, base):
        c = j // (_SPW // 16)
        ch = j % (_SPW // 16)
        boff = ch * 16 + iota16
        vals = plsc.load_gather(lab_v, [boff * C + c])
        neg = vals < 0.5
        scan = plsc.cumsum(jnp.where(neg, 1, 0))
        pos = base + scan - 1
        rowid = c * B + wbase + boff
        plsc.store_scatter(rid_v, [pos], rowid, mask=neg)
        plsc.store_scatter(bloc_v, [pos], boff, mask=neg)
        return base + jnp.max(scan)
    cnt = lax.fori_loop(0, C * (_SPW // 16), _build, 0)

    def _pad_tail(i, _):
        pp = cnt + i * 16 + iota16
        plsc.store_scatter(rid_v, [pp], zero16, mask=pp < _RPW + _PAD)
        plsc.store_scatter(bloc_v, [pp], zero16 + _SPW, mask=pp < _RPW + _PAD)
        return 0
    lax.fori_loop(0, 4, _pad_tail, 0)

    def _start(off, buf, sem):
        idxv = rid_v[pl.ds(off, 16)]
        pltpu.async_copy(tlt_hbm.at[idxv], buf, sem)

    def _wait(buf, sem):
        pltpu.make_async_copy(tlt_hbm.at[pl.ds(0, 16)], buf, sem).wait()

    def _consume(off, buf):
        acc16 = jnp.full((16,), _NEG, jnp.float32)
        for l in range(L):
            acc16 = jnp.maximum(acc16,
                                plsc.load_gather(buf, [iota16, zero16 + l]))
        rej = 1.0 / (1.0 + jnp.exp(-acc16)) - MARGIN
        rej = jnp.maximum(rej, 0.0)
        bl = bloc_v[pl.ds(off, 16)]
        plsc.addupdate_scatter(acc_v, [bl], rej)

    n2 = (cnt + 31) // 32

    _start(0, bufa, sema)
    _start(16, bufb, semb)

    def _gloop(jj, _):
        o0 = jj * 32
        _wait(bufa, sema)
        _start(o0 + 32, bufa, sema)
        _consume(o0, bufa)
        _wait(bufb, semb)
        _start(o0 + 48, bufb, semb)
        _consume(o0 + 16, bufb)
        return 0
    lax.fori_loop(0, n2, _gloop, 0)

    _wait(bufa, sema)
    _wait(bufb, semb)

    pltpu.sync_copy(acc_v.at[pl.ds(0, _SPW)], out_hbm.at[pl.ds(wbase, _SPW)])


_SC_CACHE = []


def _get_sc_rejection():
    if not _SC_CACHE:
        k = pl.kernel(
            _sc_body,
            mesh=plsc.VectorSubcoreMesh(core_axis_name="c",
                                        subcore_axis_name="s"),
            out_type=jax.ShapeDtypeStruct((B,), jnp.float32),
            compiler_params=pltpu.CompilerParams(needs_layout_passes=False),
            scratch_types=[
                pltpu.VMEM((_SPW * C,), jnp.float32),
                pltpu.VMEM((_RPW + _PAD,), jnp.int32),
                pltpu.VMEM((_RPW + _PAD,), jnp.int32),
                pltpu.VMEM((16, L), jnp.float32),
                pltpu.VMEM((16, L), jnp.float32),
                pltpu.VMEM((_SPW + 16,), jnp.float32),
                pltpu.SemaphoreType.DMA,
                pltpu.SemaphoreType.DMA,
            ],
        )
        _SC_CACHE.append(k)
    return _SC_CACHE[0]


def kernel(logits, total_cls_logits, total_cls_feature, labels, prototypes):
    logT = logits.T
    labT = labels.T
    oh = jnp.asarray(_OH)
    tc_out = _tc_call(logT, labT, total_cls_feature, prototypes, oh)
    tlt_rows = total_cls_logits.reshape(C * B, L)
    sc_out = _get_sc_rejection()(labels.reshape(B * C), tlt_rows)
    return tc_out + sc_out
